# Initial kernel scaffold; baseline (speedup 1.0000x reference)
import functools

import jax
import jax.numpy as jnp
from jax import lax
from jax.experimental import pallas as pl
from jax.experimental.pallas import tpu as pltpu
from jax.experimental.pallas import tpu_sc as plsc

N = 10000
E = 320000
D_IN = 128
D_OUT = 64
D_HID = 2 * D_OUT

NC = 2
NS = 16
NW = NC * NS
EW = E // NW
CH = 80
NCHUNK = EW // CH
RPT = N // NS

ROWS_B = 10
RB = N // ROWS_B


def _make_sc_prop(D, do_gather):
    mesh = plsc.VectorSubcoreMesh(core_axis_name="c", subcore_axis_name="s")

    @functools.partial(
        pl.kernel,
        out_type=jax.ShapeDtypeStruct((2 * N, D), jnp.float32),
        mesh=mesh,
        scratch_types=[
            pltpu.VMEM((CH,), jnp.int32),
            pltpu.VMEM((CH,), jnp.int32),
            pltpu.VMEM((CH, D), jnp.float32),
            pltpu.VMEM_SHARED((N, D), jnp.float32),
            pltpu.SemaphoreType.DMA,
        ],
    )
    def sc_prop(src_hbm, dst_hbm, t_hbm, out_hbm, sidx, didx, rows, acc, sem):
        c = lax.axis_index("c")
        s = lax.axis_index("s")
        wid = s * NC + c
        pltpu.sync_copy(t_hbm.at[pl.ds(s * RPT, RPT)], acc.at[pl.ds(s * RPT, RPT)])
        if not do_gather:
            def fill(j, carry):
                rows[j, :] = jnp.full((D,), 1.0, jnp.float32)
                return carry
            lax.fori_loop(0, CH, fill, 0)
        plsc.subcore_barrier()
        base = wid * EW

        def body(i, carry):
            off = base + i * CH
            pltpu.sync_copy(src_hbm.at[pl.ds(off, CH)], sidx)
            pltpu.sync_copy(dst_hbm.at[pl.ds(off, CH)], didx)
            if do_gather:
                pltpu.async_copy(t_hbm.at[sidx], rows, sem).wait()
            pltpu.sync_copy(rows, acc.at[didx], add=True)
            return carry

        lax.fori_loop(0, NCHUNK, body, 0)
        plsc.subcore_barrier()
        pltpu.sync_copy(
            acc.at[pl.ds(s * RPT, RPT)],
            out_hbm.at[pl.ds(c * N + s * RPT, RPT)],
        )

    return sc_prop


_sc_deg = _make_sc_prop(16, do_gather=False)
_sc_prop = _make_sc_prop(D_IN, do_gather=True)


def _tc_stage1(x, W1, dp0, dp1):
    def body(x_ref, w_ref, d0_ref, d1_ref, t1_ref, dis_ref):
        deg = d0_ref[...] + d1_ref[...] - 1.0
        dis = lax.rsqrt(deg)
        m = jnp.dot(x_ref[...], w_ref[...], preferred_element_type=jnp.float32)
        t1_ref[...] = m * dis
        dis_ref[...] = dis

    return pl.pallas_call(
        body,
        grid=(ROWS_B,),
        in_specs=[
            pl.BlockSpec((RB, D_IN), lambda i: (i, 0)),
            pl.BlockSpec((D_IN, D_HID), lambda i: (0, 0)),
            pl.BlockSpec((RB, 1), lambda i: (i, 0)),
            pl.BlockSpec((RB, 1), lambda i: (i, 0)),
        ],
        out_specs=[
            pl.BlockSpec((RB, D_HID), lambda i: (i, 0)),
            pl.BlockSpec((RB, 1), lambda i: (i, 0)),
        ],
        out_shape=[
            jax.ShapeDtypeStruct((N, D_HID), jnp.float32),
            jax.ShapeDtypeStruct((N, 1), jnp.float32),
        ],
    )(x, W1, dp0, dp1)


def _tc_stage2(pa, pb, t1, dis, b1, Wc):
    def body(pa_ref, pb_ref, t1_ref, dis_ref, b_ref, w_ref, t2_ref):
        s = pa_ref[...] + pb_ref[...] - t1_ref[...]
        h = jnp.maximum(s * dis_ref[...] + b_ref[...], 0.0)
        m = jnp.dot(h, w_ref[...], preferred_element_type=jnp.float32)
        t2_ref[...] = m * dis_ref[...]

    return pl.pallas_call(
        body,
        grid=(ROWS_B,),
        in_specs=[
            pl.BlockSpec((RB, D_HID), lambda i: (i, 0)),
            pl.BlockSpec((RB, D_HID), lambda i: (i, 0)),
            pl.BlockSpec((RB, D_HID), lambda i: (i, 0)),
            pl.BlockSpec((RB, 1), lambda i: (i, 0)),
            pl.BlockSpec((1, D_HID), lambda i: (0, 0)),
            pl.BlockSpec((D_HID, 2 * D_OUT), lambda i: (0, 0)),
        ],
        out_specs=pl.BlockSpec((RB, 2 * D_OUT), lambda i: (i, 0)),
        out_shape=jax.ShapeDtypeStruct((N, 2 * D_OUT), jnp.float32),
    )(pa, pb, t1, dis, b1, Wc)


def _tc_stage3(pa, pb, t2, dis, bmu, bls):
    def body(pa_ref, pb_ref, t2_ref, dis_ref, bm_ref, bl_ref, mu_ref, ls_ref):
        p = (pa_ref[...] + pb_ref[...] - t2_ref[...]) * dis_ref[...]
        mu_ref[...] = p[:, :D_OUT] + bm_ref[...]
        ls_ref[...] = p[:, D_OUT:] + bl_ref[...]

    return pl.pallas_call(
        body,
        grid=(ROWS_B,),
        in_specs=[
            pl.BlockSpec((RB, 2 * D_OUT), lambda i: (i, 0)),
            pl.BlockSpec((RB, 2 * D_OUT), lambda i: (i, 0)),
            pl.BlockSpec((RB, 2 * D_OUT), lambda i: (i, 0)),
            pl.BlockSpec((RB, 1), lambda i: (i, 0)),
            pl.BlockSpec((1, D_OUT), lambda i: (0, 0)),
            pl.BlockSpec((1, D_OUT), lambda i: (0, 0)),
        ],
        out_specs=[
            pl.BlockSpec((RB, D_OUT), lambda i: (i, 0)),
            pl.BlockSpec((RB, D_OUT), lambda i: (i, 0)),
        ],
        out_shape=[
            jax.ShapeDtypeStruct((N, D_OUT), jnp.float32),
            jax.ShapeDtypeStruct((N, D_OUT), jnp.float32),
        ],
    )(pa, pb, t2, dis, bmu, bls)


def kernel(x, edge_index, W1, b1, Wmu, bmu, Wls, bls):
    src = edge_index[0]
    dst = edge_index[1]
    ones16 = jnp.ones((N, 16), jnp.float32)
    Wc = jnp.concatenate([Wmu, Wls], axis=1)

    dp = _sc_deg(src, dst, ones16)
    dp0 = dp[:N, :1]
    dp1 = dp[N:, :1]

    t1, dis = _tc_stage1(x, W1, dp0, dp1)

    s1 = _sc_prop(src, dst, t1)
    t2 = _tc_stage2(s1[:N], s1[N:], t1, dis, b1.reshape(1, -1), Wc)

    s2 = _sc_prop(src, dst, t2)
    mu, ls = _tc_stage3(s2[:N], s2[N:], t2, dis,
                        bmu.reshape(1, -1), bls.reshape(1, -1))
    return (mu, ls)


# speedup vs baseline: 13.9984x; 13.9984x over previous
import functools

import jax
import jax.numpy as jnp
from jax import lax
from jax.experimental import pallas as pl
from jax.experimental.pallas import tpu as pltpu
from jax.experimental.pallas import tpu_sc as plsc

N = 10000
E = 320000
D_IN = 128
D_OUT = 64
D_HID = 2 * D_OUT

NC = 2
NS = 16
NW = NC * NS
EW = E // NW
CH = 80
NCHUNK = EW // CH
NP = 10112
RPT = NP // NS

ROWS_B = 10
RB = N // ROWS_B


def _make_sc_prop(D, do_gather):
    mesh = plsc.VectorSubcoreMesh(core_axis_name="c", subcore_axis_name="s")

    @functools.partial(
        pl.kernel,
        out_type=jax.ShapeDtypeStruct((2 * NP, D), jnp.float32),
        mesh=mesh,
        scratch_types=[
            pltpu.VMEM((CH,), jnp.int32),
            pltpu.VMEM((CH,), jnp.int32),
            pltpu.VMEM((CH, D), jnp.float32),
            pltpu.VMEM_SHARED((NP, D), jnp.float32),
            pltpu.SemaphoreType.DMA,
        ],
    )
    def sc_prop(src_hbm, dst_hbm, t_hbm, out_hbm, sidx, didx, rows, acc, sem):
        c = lax.axis_index("c")
        s = lax.axis_index("s")
        wid = s * NC + c
        pltpu.sync_copy(t_hbm.at[pl.ds(s * RPT, RPT)], acc.at[pl.ds(s * RPT, RPT)])
        if not do_gather:
            def fill(j, carry):
                rows[j, :] = jnp.full((D,), 1.0, jnp.float32)
                return carry
            lax.fori_loop(0, CH, fill, 0)
        plsc.subcore_barrier()
        base = wid * EW

        def body(i, carry):
            off = base + i * CH
            pltpu.sync_copy(src_hbm.at[pl.ds(off, CH)], sidx)
            pltpu.sync_copy(dst_hbm.at[pl.ds(off, CH)], didx)
            if do_gather:
                pltpu.async_copy(t_hbm.at[sidx], rows, sem).wait()
            pltpu.sync_copy(rows, acc.at[didx], add=True)
            return carry

        lax.fori_loop(0, NCHUNK, body, 0)
        plsc.subcore_barrier()
        pltpu.sync_copy(
            acc.at[pl.ds(s * RPT, RPT)],
            out_hbm.at[pl.ds(c * NP + s * RPT, RPT)],
        )

    return sc_prop


_sc_deg = _make_sc_prop(16, do_gather=False)
_sc_prop = _make_sc_prop(D_IN, do_gather=True)


def _tc_stage1(x, W1, dp0, dp1):
    def body(x_ref, w_ref, d0_ref, d1_ref, t1_ref, dis_ref):
        deg = d0_ref[...] + d1_ref[...] - 1.0
        dis = lax.rsqrt(deg)
        m = jnp.dot(x_ref[...], w_ref[...], preferred_element_type=jnp.float32)
        t1_ref[...] = m * dis
        dis_ref[...] = dis

    return pl.pallas_call(
        body,
        grid=(ROWS_B,),
        in_specs=[
            pl.BlockSpec((RB, D_IN), lambda i: (i, 0)),
            pl.BlockSpec((D_IN, D_HID), lambda i: (0, 0)),
            pl.BlockSpec((RB, 1), lambda i: (i, 0)),
            pl.BlockSpec((RB, 1), lambda i: (i, 0)),
        ],
        out_specs=[
            pl.BlockSpec((RB, D_HID), lambda i: (i, 0)),
            pl.BlockSpec((RB, 1), lambda i: (i, 0)),
        ],
        out_shape=[
            jax.ShapeDtypeStruct((N, D_HID), jnp.float32),
            jax.ShapeDtypeStruct((N, 1), jnp.float32),
        ],
    )(x, W1, dp0, dp1)


def _tc_stage2(pa, pb, t1, dis, b1, Wc):
    def body(pa_ref, pb_ref, t1_ref, dis_ref, b_ref, w_ref, t2_ref):
        s = pa_ref[...] + pb_ref[...] - t1_ref[...]
        h = jnp.maximum(s * dis_ref[...] + b_ref[...], 0.0)
        m = jnp.dot(h, w_ref[...], preferred_element_type=jnp.float32)
        t2_ref[...] = m * dis_ref[...]

    return pl.pallas_call(
        body,
        grid=(ROWS_B,),
        in_specs=[
            pl.BlockSpec((RB, D_HID), lambda i: (i, 0)),
            pl.BlockSpec((RB, D_HID), lambda i: (i, 0)),
            pl.BlockSpec((RB, D_HID), lambda i: (i, 0)),
            pl.BlockSpec((RB, 1), lambda i: (i, 0)),
            pl.BlockSpec((1, D_HID), lambda i: (0, 0)),
            pl.BlockSpec((D_HID, 2 * D_OUT), lambda i: (0, 0)),
        ],
        out_specs=pl.BlockSpec((RB, 2 * D_OUT), lambda i: (i, 0)),
        out_shape=jax.ShapeDtypeStruct((N, 2 * D_OUT), jnp.float32),
    )(pa, pb, t1, dis, b1, Wc)


def _tc_stage3(pa, pb, t2, dis, bmu, bls):
    def body(pa_ref, pb_ref, t2_ref, dis_ref, bm_ref, bl_ref, mu_ref, ls_ref):
        p = (pa_ref[...] + pb_ref[...] - t2_ref[...]) * dis_ref[...]
        mu_ref[...] = p[:, :D_OUT] + bm_ref[...]
        ls_ref[...] = p[:, D_OUT:] + bl_ref[...]

    return pl.pallas_call(
        body,
        grid=(ROWS_B,),
        in_specs=[
            pl.BlockSpec((RB, 2 * D_OUT), lambda i: (i, 0)),
            pl.BlockSpec((RB, 2 * D_OUT), lambda i: (i, 0)),
            pl.BlockSpec((RB, 2 * D_OUT), lambda i: (i, 0)),
            pl.BlockSpec((RB, 1), lambda i: (i, 0)),
            pl.BlockSpec((1, D_OUT), lambda i: (0, 0)),
            pl.BlockSpec((1, D_OUT), lambda i: (0, 0)),
        ],
        out_specs=[
            pl.BlockSpec((RB, D_OUT), lambda i: (i, 0)),
            pl.BlockSpec((RB, D_OUT), lambda i: (i, 0)),
        ],
        out_shape=[
            jax.ShapeDtypeStruct((N, D_OUT), jnp.float32),
            jax.ShapeDtypeStruct((N, D_OUT), jnp.float32),
        ],
    )(pa, pb, t2, dis, bmu, bls)


def kernel(x, edge_index, W1, b1, Wmu, bmu, Wls, bls):
    src = edge_index[0]
    dst = edge_index[1]
    ones16 = jnp.ones((NP, 16), jnp.float32)
    Wc = jnp.concatenate([Wmu, Wls], axis=1)

    dp = _sc_deg(src, dst, ones16)
    dp0 = dp[:N, :1]
    dp1 = dp[NP:NP + N, :1]

    t1, dis = _tc_stage1(x, W1, dp0, dp1)

    t1p = jnp.pad(t1, ((0, NP - N), (0, 0)))
    s1 = _sc_prop(src, dst, t1p)
    t2 = _tc_stage2(s1[:N], s1[NP:NP + N], t1, dis, b1.reshape(1, -1), Wc)

    t2p = jnp.pad(t2, ((0, NP - N), (0, 0)))
    s2 = _sc_prop(src, dst, t2p)
    mu, ls = _tc_stage3(s2[:N], s2[NP:NP + N], t2, dis,
                        bmu.reshape(1, -1), bls.reshape(1, -1))
    return (mu, ls)
